# Initial kernel scaffold; baseline (speedup 1.0000x reference)
import jax
import jax.numpy as jnp
from jax.experimental import pallas as pl


def kernel(pos, atom_idx, aa_idx, sake_edges, schnet_edges, params):
    raise NotImplementedError("write your pallas kernel here")


# speedup vs baseline: 1.1783x; 1.1783x over previous
import functools

import jax
import jax.numpy as jnp
from jax.experimental import pallas as pl
from jax.experimental.pallas import tpu as pltpu

N = 50000
E = 400000
H = 64
NH = 4
HD = 16

BLK_E = 4000
GE = E // BLK_E
BLK_N = 2000
GN = N // BLK_N


def _celu(x):
    return jnp.where(x > 0, x, 2.0 * (jnp.exp(x * 0.5) - 1.0))


def _row_spec(*blk):
    nd = len(blk)
    return pl.BlockSpec(blk, lambda i: (i,) + (0,) * (nd - 1))


def _full_spec(*shape):
    nd = len(shape)
    return pl.BlockSpec(shape, lambda i: (0,) * nd)


def _head_expand():
    lane = jax.lax.broadcasted_iota(jnp.int32, (NH, H), 1)
    sub = jax.lax.broadcasted_iota(jnp.int32, (NH, H), 0)
    return (lane // HD == sub).astype(jnp.float32)


def _node0_body(aa_ref, at_ref, tab_ref, wa_ref, wb_ref, z0_ref, zs_ref, zd_ref):
    aa = aa_ref[0, 0, :]
    at = at_ref[0, 0, :]
    io = jax.lax.broadcasted_iota(jnp.int32, (BLK_N, 24), 1)
    oh = (aa[:, None] == io).astype(jnp.float32) + ((at[:, None] + 21) == io).astype(jnp.float32)
    z0 = oh @ tab_ref[...]
    z0_ref[...] = z0
    zs_ref[...] = z0 @ wa_ref[...]
    zd_ref[...] = z0 @ wb_ref[...]


def _geo_body(sd_ref, cd_ref, dir_ref, srbf_ref, crbf_ref):
    io24 = jax.lax.broadcasted_iota(jnp.int32, (1, 24), 1).astype(jnp.float32)
    sd = sd_ref[...]
    d2 = jnp.sum(sd * sd, axis=1, keepdims=True)
    rad = jnp.sqrt(d2 + 1e-12)
    dir_ref[...] = (sd * (1.0 / (rad + 1e-9)))[:, :8]
    c_s = 0.25 + io24 * (0.75 / 17.0)
    g_s = 1.0 / ((0.75 / 18.0) ** 2)
    srbf_ref[...] = jnp.exp(-g_s * (rad - c_s) ** 2)
    cd = cd_ref[...]
    cd2 = jnp.sum(cd * cd, axis=1, keepdims=True)
    crad = jnp.sqrt(cd2 + 1e-12)
    c_c = 1.0 + io24 * (1.5 / 17.0)
    g_c = 1.0 / ((1.5 / 18.0) ** 2)
    crbf_ref[...] = jnp.exp(-g_c * (crad - c_c) ** 2)


def _edgeA_body(zsum_ref, rbf_ref, wc_ref, bm_ref, wag_ref, bag_ref,
                e_ref, lg_ref, bmax_ref):
    pre = zsum_ref[...] + rbf_ref[...] @ wc_ref[...] + bm_ref[...]
    e = _celu(pre)
    lg = e @ wag_ref[...] + bag_ref[...]
    e_ref[...] = e
    lg_ref[...] = lg
    bmax_ref[...] = jnp.max(lg, axis=0, keepdims=True)[None]


def _edgeB_body(e_ref, lg_ref, dir_ref, bmax_ref, p0_ref, p1_ref):
    g = jnp.max(bmax_ref[...], axis=0)
    lg = lg_ref[...]
    ex = jnp.exp(lg[:, :4] - g[:, :4])
    exr = ex @ _head_expand()
    w = e_ref[...] * exr
    gate = lg[:, 4:5]
    gdir = dir_ref[...][:, :3] * gate
    p0_ref[...] = jnp.concatenate(
        [w[:, :32], ex, gdir, jnp.zeros((BLK_E, 1), jnp.float32)], axis=1)
    p1_ref[...] = w[:, 32:]


def _nodeC_body(z_ref, a0_ref, a1_ref, wuz_ref, wua_ref, wuv_ref, bu_ref,
                wl1_ref, bl1_ref, zmid_ref, z1_ref):
    a0 = a0_ref[...]
    num = jnp.concatenate([a0[:, :32], a1_ref[...]], axis=1)
    den = a0[:, 32:36] @ _head_expand()
    agg = jnp.where(den > 0, num / den, 0.0)
    vec = a0[:, 36:39]
    vn = jnp.sqrt(jnp.sum(vec * vec, axis=1, keepdims=True) + 1e-9)
    z = z_ref[...]
    u = z @ wuz_ref[...] + agg @ wua_ref[...] + vn * wuv_ref[...] + bu_ref[...]
    zmid = z + _celu(u)
    zmid_ref[...] = zmid
    z1_ref[...] = zmid @ wl1_ref[...] + bl1_ref[...]


def _edgeS_body(z1g_ref, rbf_ref, wf1_ref, bf1_ref, wf2_ref, bf2_ref,
                m0_ref, m1_ref):
    w = _celu(rbf_ref[...] @ wf1_ref[...] + bf1_ref[...]) @ wf2_ref[...] + bf2_ref[...]
    m = z1g_ref[...] * w
    m0_ref[...] = m[:, :32]
    m1_ref[...] = m[:, 32:]


def _nodeE_body(zmid_ref, a0_ref, a1_ref, wl2_ref, bl2_ref, wl3_ref, bl3_ref,
                wa_ref, wb_ref, z_ref, zs_ref, zd_ref):
    agg2 = jnp.concatenate([a0_ref[...], a1_ref[...]], axis=1)
    z = zmid_ref[...] + _celu(agg2 @ wl2_ref[...] + bl2_ref[...]) @ wl3_ref[...] + bl3_ref[...]
    z_ref[...] = z
    zs_ref[...] = z @ wa_ref[...]
    zd_ref[...] = z @ wb_ref[...]


def _nodeF_body(zmid_ref, a0_ref, a1_ref, wl2_ref, bl2_ref, wl3_ref, bl3_ref,
                we_ref, be_ref, w1_ref, b1_ref, w2_ref, b2_ref, w3_ref, b3_ref,
                out_ref):
    agg2 = jnp.concatenate([a0_ref[...], a1_ref[...]], axis=1)
    z = zmid_ref[...] + _celu(agg2 @ wl2_ref[...] + bl2_ref[...]) @ wl3_ref[...] + bl3_ref[...]
    zo = z @ we_ref[...] + be_ref[...]
    h = jnp.tanh(zo @ w1_ref[...] + b1_ref[...])
    h = jnp.tanh(h @ w2_ref[...] + b2_ref[...])
    out_ref[...] = h @ w3_ref[...] + b3_ref[...]


def _f32(*shape):
    return jax.ShapeDtypeStruct(shape, jnp.float32)


def _node0_call(aa3, at3, tab, wa, wb):
    return pl.pallas_call(
        _node0_body, grid=(GN,),
        in_specs=[_row_spec(1, 1, BLK_N), _row_spec(1, 1, BLK_N),
                  _full_spec(24, H), _full_spec(H, H), _full_spec(H, H)],
        out_specs=[_row_spec(BLK_N, H)] * 3,
        out_shape=[_f32(N, H)] * 3,
    )(aa3, at3, tab, wa, wb)


def _geo_call(sdiff, cdiff):
    return pl.pallas_call(
        _geo_body, grid=(GE,),
        in_specs=[_row_spec(BLK_E, 16), _row_spec(BLK_E, 16)],
        out_specs=[_row_spec(BLK_E, 8), _row_spec(BLK_E, 24), _row_spec(BLK_E, 24)],
        out_shape=[_f32(E, 8), _f32(E, 24), _f32(E, 24)],
    )(sdiff, cdiff)


def _edgeA_call(zsum, srbf, wc, bm, wag, bag):
    return pl.pallas_call(
        _edgeA_body, grid=(GE,),
        in_specs=[_row_spec(BLK_E, H), _row_spec(BLK_E, 24), _full_spec(24, H),
                  _full_spec(1, H), _full_spec(H, 8), _full_spec(1, 8)],
        out_specs=[_row_spec(BLK_E, H), _row_spec(BLK_E, 8), _row_spec(1, 1, 8)],
        out_shape=[_f32(E, H), _f32(E, 8), _f32(GE, 1, 8)],
    )(zsum, srbf, wc, bm, wag, bag)


def _edgeB_call(e, lg, dir8, bmax):
    return pl.pallas_call(
        _edgeB_body, grid=(GE,),
        in_specs=[_row_spec(BLK_E, H), _row_spec(BLK_E, 8), _row_spec(BLK_E, 8),
                  _full_spec(GE, 1, 8)],
        out_specs=[_row_spec(BLK_E, 40), _row_spec(BLK_E, 32)],
        out_shape=[_f32(E, 40), _f32(E, 32)],
    )(e, lg, dir8, bmax)


def _nodeC_call(z, a0, a1, wuz, wua, wuv, bu, wl1, bl1):
    return pl.pallas_call(
        _nodeC_body, grid=(GN,),
        in_specs=[_row_spec(BLK_N, H), _row_spec(BLK_N, 40), _row_spec(BLK_N, 32),
                  _full_spec(H, H), _full_spec(H, H), _full_spec(1, H),
                  _full_spec(1, H), _full_spec(H, H), _full_spec(1, H)],
        out_specs=[_row_spec(BLK_N, H)] * 2,
        out_shape=[_f32(N, H)] * 2,
    )(z, a0, a1, wuz, wua, wuv, bu, wl1, bl1)


def _edgeS_call(z1g, crbf, wf1, bf1, wf2, bf2):
    return pl.pallas_call(
        _edgeS_body, grid=(GE,),
        in_specs=[_row_spec(BLK_E, H), _row_spec(BLK_E, 24), _full_spec(24, H),
                  _full_spec(1, H), _full_spec(H, H), _full_spec(1, H)],
        out_specs=[_row_spec(BLK_E, 32), _row_spec(BLK_E, 32)],
        out_shape=[_f32(E, 32), _f32(E, 32)],
    )(z1g, crbf, wf1, bf1, wf2, bf2)


def _nodeE_call(zmid, a0, a1, wl2, bl2, wl3, bl3, wa, wb):
    return pl.pallas_call(
        _nodeE_body, grid=(GN,),
        in_specs=[_row_spec(BLK_N, H), _row_spec(BLK_N, 32), _row_spec(BLK_N, 32),
                  _full_spec(H, H), _full_spec(1, H), _full_spec(H, H),
                  _full_spec(1, H), _full_spec(H, H), _full_spec(H, H)],
        out_specs=[_row_spec(BLK_N, H)] * 3,
        out_shape=[_f32(N, H)] * 3,
    )(zmid, a0, a1, wl2, bl2, wl3, bl3, wa, wb)


def _nodeF_call(zmid, a0, a1, wl2, bl2, wl3, bl3, we, be, w1, b1, w2, b2, w3, b3):
    return pl.pallas_call(
        _nodeF_body, grid=(GN,),
        in_specs=[_row_spec(BLK_N, H), _row_spec(BLK_N, 32), _row_spec(BLK_N, 32),
                  _full_spec(H, H), _full_spec(1, H), _full_spec(H, H),
                  _full_spec(1, H), _full_spec(H, H), _full_spec(1, H),
                  _full_spec(H, H), _full_spec(1, H), _full_spec(H, H),
                  _full_spec(1, H), _full_spec(H, 8), _full_spec(1, 8)],
        out_specs=[_row_spec(BLK_N, 8)],
        out_shape=[_f32(N, 8)],
    )(zmid, a0, a1, wl2, bl2, wl3, bl3, we, be, w1, b1, w2, b2, w3, b3)


def kernel(pos, atom_idx, aa_idx, sake_edges, schnet_edges, params):
    f32 = jnp.float32
    atom_map = jnp.array([63, 1, 0], dtype=jnp.int32)
    aa_map = jnp.concatenate([jnp.arange(20, dtype=jnp.int32),
                              jnp.array([0], dtype=jnp.int32)])
    tab = jnp.zeros((24, H), f32)
    tab = tab.at[:21, :32].set(params["aa_embed"][aa_map])
    tab = tab.at[21:24, 32:].set(params["atom_embed"][atom_map])

    def lin(p):
        return p["w"].astype(f32), p["b"].astype(f32)

    LW = []
    for lay in params["layers"]:
        wm, bm = lin(lay["sake_msg"])
        wa, ba = lin(lay["sake_att"])
        wg, bg = lin(lay["sake_gate"])
        wu, bu = lin(lay["sake_upd"])
        wf1, bf1 = lin(lay["schnet_f1"])
        wf2, bf2 = lin(lay["schnet_f2"])
        wl1, bl1 = lin(lay["schnet_l1"])
        wl2, bl2 = lin(lay["schnet_l2"])
        wl3, bl3 = lin(lay["schnet_l3"])
        wc24 = jnp.zeros((24, H), f32).at[:18].set(wm[128:146])
        wag = jnp.zeros((H, 8), f32).at[:, :4].set(wa).at[:, 4:5].set(wg)
        bag = jnp.zeros((1, 8), f32).at[0, :4].set(ba).at[0, 4].set(bg[0])
        LW.append(dict(
            wa=wm[:64], wb=wm[64:128], wc=wc24, bm=bm[None], wag=wag, bag=bag,
            wuz=wu[:64], wua=wu[64:128], wuv=wu[128:129], bu=bu[None],
            wf1=jnp.zeros((24, H), f32).at[:18].set(wf1), bf1=bf1[None],
            wf2=wf2, bf2=bf2[None],
            wl1=wl1, bl1=bl1[None], wl2=wl2, bl2=bl2[None], wl3=wl3, bl3=bl3[None],
        ))
    we, be = lin(params["embed_out"])
    w1, b1 = lin(params["out"][0])
    w2, b2 = lin(params["out"][1])
    w3, b3 = lin(params["out"][2])
    w3p = jnp.zeros((H, 8), f32).at[:, :1].set(w3)
    b3p = jnp.zeros((1, 8), f32).at[0, 0].set(b3[0])

    s_src, s_dst = sake_edges[0], sake_edges[1]
    c_src, c_dst = schnet_edges[0], schnet_edges[1]

    aa3 = aa_idx.astype(jnp.int32).reshape(GN, 1, BLK_N)
    at3 = atom_idx.astype(jnp.int32).reshape(GN, 1, BLK_N)

    pos16 = jnp.pad(pos, ((0, 0), (0, 13)))
    sdiff = jnp.take(pos16, s_src, axis=0) - jnp.take(pos16, s_dst, axis=0)
    cdiff = jnp.take(pos16, c_src, axis=0) - jnp.take(pos16, c_dst, axis=0)
    dir8, srbf, crbf = _geo_call(sdiff, cdiff)

    z, zs, zd = _node0_call(aa3, at3, tab, LW[0]["wa"], LW[0]["wb"])

    for li, W in enumerate(LW):
        zsum = jnp.take(zs, s_src, axis=0) + jnp.take(zd, s_dst, axis=0)
        e, lg, bmax = _edgeA_call(zsum, srbf, W["wc"], W["bm"], W["wag"], W["bag"])
        p0, p1 = _edgeB_call(e, lg, dir8, bmax)
        acc0 = jax.ops.segment_sum(p0, s_dst, num_segments=N)
        acc1 = jax.ops.segment_sum(p1, s_dst, num_segments=N)
        zmid, z1 = _nodeC_call(z, acc0, acc1, W["wuz"], W["wua"], W["wuv"],
                               W["bu"], W["wl1"], W["bl1"])
        z1g = jnp.take(z1, c_src, axis=0)
        m0, m1 = _edgeS_call(z1g, crbf, W["wf1"], W["bf1"], W["wf2"], W["bf2"])
        b0 = jax.ops.segment_sum(m0, c_dst, num_segments=N)
        b1_ = jax.ops.segment_sum(m1, c_dst, num_segments=N)
        if li + 1 < len(LW):
            W2 = LW[li + 1]
            z, zs, zd = _nodeE_call(zmid, b0, b1_, W["wl2"], W["bl2"],
                                    W["wl3"], W["bl3"], W2["wa"], W2["wb"])
        else:
            [out8] = _nodeF_call(zmid, b0, b1_, W["wl2"], W["bl2"], W["wl3"],
                               W["bl3"], we, be[None], w1, b1[None], w2,
                               b2[None], w3p, b3p)
    return out8[:, :1]


# speedup vs baseline: 1.9290x; 1.6371x over previous
import functools

import jax
import jax.numpy as jnp
from jax import lax
from jax.experimental import pallas as pl
from jax.experimental.pallas import tpu as pltpu
from jax.experimental.pallas import tpu_sc as plsc

N = 50000
E = 400000
H = 64
NH = 4
HD = 16

EP = 425984
WIN = 512
NWORK = 32
CH_G = EP // NWORK
N_ACC = 50176

BLK_E = 4096
GE = EP // BLK_E
BLK_N = 2000
GN = N // BLK_N


def _celu(x):
    return jnp.where(x > 0, x, 2.0 * (jnp.exp(x * 0.5) - 1.0))


def _row_spec(*blk):
    nd = len(blk)
    return pl.BlockSpec(blk, lambda i: (i,) + (0,) * (nd - 1))


def _full_spec(*shape):
    nd = len(shape)
    return pl.BlockSpec(shape, lambda i: (0,) * nd)


def _head_expand():
    lane = jax.lax.broadcasted_iota(jnp.int32, (NH, H), 1)
    sub = jax.lax.broadcasted_iota(jnp.int32, (NH, H), 0)
    return (lane // HD == sub).astype(jnp.float32)


def _rbf24(rad, low, high):
    io24 = jax.lax.broadcasted_iota(jnp.int32, (1, 24), 1).astype(jnp.float32)
    centers = low + io24 * ((high - low) / 17.0)
    gamma = 1.0 / (((high - low) / 18.0) ** 2)
    return jnp.exp(-gamma * (rad - centers) ** 2)


def _sc_mesh():
    return plsc.VectorSubcoreMesh(core_axis_name="c", subcore_axis_name="s")


def _m8(x):
    return pl.multiple_of(x, 8)


def _gathersum_call(tabA, tabB, ia2, ib2, smap):
    npair = CH_G // 1024

    @functools.partial(
        pl.kernel, mesh=_sc_mesh(),
        out_type=jax.ShapeDtypeStruct((EP, 128), jnp.float32),
        scratch_types=[
            pltpu.VMEM((8, 128), jnp.int32),
            pltpu.VMEM((8, 128), jnp.int32),
            pltpu.VMEM((128, 128), jnp.float32),
            pltpu.VMEM((128, 128), jnp.float32),
            pltpu.VMEM((WIN, 128), jnp.float32),
            pltpu.SemaphoreType.DMA,
        ],
    )
    def k(tabA_h, tabB_h, ia_h, ib_h, out_h, ia_v, ib_v, av, bv, ov, sem):
        wid = lax.axis_index("s") * 2 + lax.axis_index("c")
        base = wid * CH_G

        def pair_body(w, carry):
            b0 = base + w * 1024
            pltpu.sync_copy(ia_h.at[pl.ds(_m8(b0 // 128), 8)], ia_v)
            pltpu.sync_copy(ib_h.at[pl.ds(_m8(b0 // 128), 8)], ib_v)
            for half in range(2):
                hb = b0 + half * WIN
                for j in range(4):
                    r = half * 4 + j
                    h1 = pltpu.async_copy(tabA_h.at[ia_v.at[r]], av, sem)
                    h2 = pltpu.async_copy(tabB_h.at[ib_v.at[r]], bv, sem)
                    h1.wait()
                    h2.wait()

                    def mix_body(r8, carry2):
                        for i in range(8):
                            rr = r8 * 8 + i
                            for (oo, ao, bo) in smap:
                                va = av[rr, pl.ds(ao, 16)]
                                if bo is not None:
                                    va = va + bv[rr, pl.ds(bo, 16)]
                                ov[j * 128 + rr, pl.ds(oo, 16)] = va
                        return carry2

                    lax.fori_loop(0, 16, mix_body, 0)
                pltpu.sync_copy(ov, out_h.at[pl.ds(_m8(hb), WIN)])
            return carry

        lax.fori_loop(0, npair, pair_body, 0)

    return k(tabA, tabB, ia2, ib2)


def _node0_body(aa_ref, at_ref, tab_ref, wa_ref, wb_ref, pn_ref,
                z0_ref, ta_ref, tb_ref):
    aa = aa_ref[0, 0, :]
    at = at_ref[0, 0, :]
    io = jax.lax.broadcasted_iota(jnp.int32, (BLK_N, 24), 1)
    oh = (aa[:, None] == io).astype(jnp.float32) + ((at[:, None] + 21) == io).astype(jnp.float32)
    z0 = oh @ tab_ref[...]
    z0_ref[...] = z0
    pn = pn_ref[...]
    ta_ref[...] = jnp.concatenate([z0 @ wa_ref[...], pn], axis=1)
    tb_ref[...] = jnp.concatenate([z0 @ wb_ref[...], pn], axis=1)


def _edgeA_body(zsg_ref, wc_ref, bm_ref, wag_ref, bag_ref, elgd_ref, bmax_ref):
    zsg = zsg_ref[...]
    zsum = zsg[:, :64]
    sd = zsg[:, 64:80]
    d2 = jnp.sum(sd * sd, axis=1, keepdims=True)
    rad = jnp.sqrt(d2 + 1e-12)
    dir3 = (sd * (1.0 / (rad + 1e-9)))[:, :3]
    rbf = _rbf24(rad, 0.25, 1.0)
    pre = zsum + rbf @ wc_ref[...] + bm_ref[...]
    e = _celu(pre)
    lg = e @ wag_ref[...] + bag_ref[...]
    zp = jnp.zeros((BLK_E, 128 - 75), jnp.float32)
    elgd_ref[...] = jnp.concatenate([e, lg, dir3, zp], axis=1)
    bmax_ref[...] = jnp.max(lg, axis=0, keepdims=True)[None]


def _edgeB_body(elgd_ref, bmax_ref, p0_ref, p1_ref):
    g = jnp.max(bmax_ref[...], axis=0)
    elgd = elgd_ref[...]
    e = elgd[:, :64]
    lg = elgd[:, 64:72]
    ex = jnp.exp(lg[:, :4] - g[:, :4])
    exr = ex @ _head_expand()
    w = e * exr
    gate = lg[:, 4:5]
    gdir = elgd[:, 72:75] * gate
    zpad = jnp.zeros((BLK_E, 1), jnp.float32)
    p0_ref[...] = jnp.concatenate([w[:, :32], ex], axis=1)
    p1_ref[...] = jnp.concatenate([w[:, 32:], gdir, zpad], axis=1)


def _nodeC_body(z_ref, a0_ref, a1_ref, wuz_ref, wua_ref, wuv_ref, bu_ref,
                wl1_ref, bl1_ref, pn_ref, zmid_ref, t1_ref):
    a0 = a0_ref[...]
    a1 = a1_ref[...]
    num = jnp.concatenate([a0[:, :32], a1[:, :32]], axis=1)
    den = a0[:, 32:36] @ _head_expand()
    agg = jnp.where(den > 0, num / den, 0.0)
    vec = a1[:, 32:35]
    vn = jnp.sqrt(jnp.sum(vec * vec, axis=1, keepdims=True) + 1e-9)
    z = z_ref[...]
    u = z @ wuz_ref[...] + agg @ wua_ref[...] + vn * wuv_ref[...] + bu_ref[...]
    zmid = z + _celu(u)
    zmid_ref[...] = zmid
    z1 = zmid @ wl1_ref[...] + bl1_ref[...]
    t1_ref[...] = jnp.concatenate([z1, pn_ref[...]], axis=1)


def _edgeS_body(zc_ref, wf1_ref, bf1_ref, wf2_ref, bf2_ref, m0_ref, m1_ref):
    zc = zc_ref[...]
    z1g = zc[:, :64]
    cd = zc[:, 64:80]
    cd2 = jnp.sum(cd * cd, axis=1, keepdims=True)
    crad = jnp.sqrt(cd2 + 1e-12)
    rbf = _rbf24(crad, 1.0, 2.5)
    w = _celu(rbf @ wf1_ref[...] + bf1_ref[...]) @ wf2_ref[...] + bf2_ref[...]
    m = z1g * w
    zp4 = jnp.zeros((BLK_E, 4), jnp.float32)
    m0_ref[...] = jnp.concatenate([m[:, :32], zp4], axis=1)
    m1_ref[...] = jnp.concatenate([m[:, 32:], zp4], axis=1)


def _nodeE_body(zmid_ref, a0_ref, a1_ref, wl2_ref, bl2_ref, wl3_ref, bl3_ref,
                wa_ref, wb_ref, pn_ref, z_ref, ta_ref, tb_ref):
    agg2 = jnp.concatenate([a0_ref[...][:, :32], a1_ref[...][:, :32]], axis=1)
    z = zmid_ref[...] + _celu(agg2 @ wl2_ref[...] + bl2_ref[...]) @ wl3_ref[...] + bl3_ref[...]
    z_ref[...] = z
    pn = pn_ref[...]
    ta_ref[...] = jnp.concatenate([z @ wa_ref[...], pn], axis=1)
    tb_ref[...] = jnp.concatenate([z @ wb_ref[...], pn], axis=1)


def _nodeF_body(zmid_ref, a0_ref, a1_ref, wl2_ref, bl2_ref, wl3_ref, bl3_ref,
                we_ref, be_ref, w1_ref, b1_ref, w2_ref, b2_ref, w3_ref, b3_ref,
                out_ref):
    agg2 = jnp.concatenate([a0_ref[...][:, :32], a1_ref[...][:, :32]], axis=1)
    z = zmid_ref[...] + _celu(agg2 @ wl2_ref[...] + bl2_ref[...]) @ wl3_ref[...] + bl3_ref[...]
    zo = z @ we_ref[...] + be_ref[...]
    h = jnp.tanh(zo @ w1_ref[...] + b1_ref[...])
    h = jnp.tanh(h @ w2_ref[...] + b2_ref[...])
    out_ref[...] = h @ w3_ref[...] + b3_ref[...]


def _f32(*shape):
    return jax.ShapeDtypeStruct(shape, jnp.float32)


def _node0_call(aa3, at3, tab, wa, wb, pn):
    return pl.pallas_call(
        _node0_body, grid=(GN,),
        in_specs=[_row_spec(1, 1, BLK_N), _row_spec(1, 1, BLK_N),
                  _full_spec(24, H), _full_spec(H, H), _full_spec(H, H),
                  _row_spec(BLK_N, H)],
        out_specs=[_row_spec(BLK_N, H), _row_spec(BLK_N, 128), _row_spec(BLK_N, 128)],
        out_shape=[_f32(N, H), _f32(N, 128), _f32(N, 128)],
    )(aa3, at3, tab, wa, wb, pn)


def _edgeA_call(zsg, wc, bm, wag, bag):
    return pl.pallas_call(
        _edgeA_body, grid=(GE,),
        in_specs=[_row_spec(BLK_E, 128), _full_spec(24, H),
                  _full_spec(1, H), _full_spec(H, 8), _full_spec(1, 8)],
        out_specs=[_row_spec(BLK_E, 128), _row_spec(1, 1, 8)],
        out_shape=[_f32(EP, 128), _f32(GE, 1, 8)],
    )(zsg, wc, bm, wag, bag)


def _edgeB_call(elgd, bmax):
    return pl.pallas_call(
        _edgeB_body, grid=(GE,),
        in_specs=[_row_spec(BLK_E, 128), _full_spec(GE, 1, 8)],
        out_specs=[_row_spec(BLK_E, 36), _row_spec(BLK_E, 36)],
        out_shape=[_f32(EP, 36), _f32(EP, 36)],
    )(elgd, bmax)


def _nodeC_call(z, a0, a1, wuz, wua, wuv, bu, wl1, bl1, pn):
    return pl.pallas_call(
        _nodeC_body, grid=(GN,),
        in_specs=[_row_spec(BLK_N, H), _row_spec(BLK_N, 36), _row_spec(BLK_N, 36),
                  _full_spec(H, H), _full_spec(H, H), _full_spec(1, H),
                  _full_spec(1, H), _full_spec(H, H), _full_spec(1, H),
                  _row_spec(BLK_N, H)],
        out_specs=[_row_spec(BLK_N, H), _row_spec(BLK_N, 128)],
        out_shape=[_f32(N, H), _f32(N, 128)],
    )(z, a0, a1, wuz, wua, wuv, bu, wl1, bl1, pn)


def _edgeS_call(zc, wf1, bf1, wf2, bf2):
    return pl.pallas_call(
        _edgeS_body, grid=(GE,),
        in_specs=[_row_spec(BLK_E, 128), _full_spec(24, H),
                  _full_spec(1, H), _full_spec(H, H), _full_spec(1, H)],
        out_specs=[_row_spec(BLK_E, 36), _row_spec(BLK_E, 36)],
        out_shape=[_f32(EP, 36), _f32(EP, 36)],
    )(zc, wf1, bf1, wf2, bf2)


def _nodeE_call(zmid, a0, a1, wl2, bl2, wl3, bl3, wa, wb, pn):
    return pl.pallas_call(
        _nodeE_body, grid=(GN,),
        in_specs=[_row_spec(BLK_N, H), _row_spec(BLK_N, 36), _row_spec(BLK_N, 36),
                  _full_spec(H, H), _full_spec(1, H), _full_spec(H, H),
                  _full_spec(1, H), _full_spec(H, H), _full_spec(H, H),
                  _row_spec(BLK_N, H)],
        out_specs=[_row_spec(BLK_N, H), _row_spec(BLK_N, 128), _row_spec(BLK_N, 128)],
        out_shape=[_f32(N, H), _f32(N, 128), _f32(N, 128)],
    )(zmid, a0, a1, wl2, bl2, wl3, bl3, wa, wb, pn)


def _nodeF_call(zmid, a0, a1, wl2, bl2, wl3, bl3, we, be, w1, b1, w2, b2, w3, b3):
    return pl.pallas_call(
        _nodeF_body, grid=(GN,),
        in_specs=[_row_spec(BLK_N, H), _row_spec(BLK_N, 36), _row_spec(BLK_N, 36),
                  _full_spec(H, H), _full_spec(1, H), _full_spec(H, H),
                  _full_spec(1, H), _full_spec(H, H), _full_spec(1, H),
                  _full_spec(H, H), _full_spec(1, H), _full_spec(H, H),
                  _full_spec(1, H), _full_spec(H, 8), _full_spec(1, 8)],
        out_specs=[_row_spec(BLK_N, 8)],
        out_shape=[_f32(N, 8)],
    )(zmid, a0, a1, wl2, bl2, wl3, bl3, we, be, w1, b1, w2, b2, w3, b3)


_SMAP_SAKE = [(0, 0, 0), (16, 16, 16), (32, 32, 32), (48, 48, 48), (64, 64, 80)]
_SMAP_SCHNET = [(0, 0, None), (16, 16, None), (32, 32, None), (48, 48, None),
                (64, 64, 80)]


def kernel(pos, atom_idx, aa_idx, sake_edges, schnet_edges, params):
    f32 = jnp.float32
    atom_map = jnp.array([63, 1, 0], dtype=jnp.int32)
    aa_map = jnp.concatenate([jnp.arange(20, dtype=jnp.int32),
                              jnp.array([0], dtype=jnp.int32)])
    tab = jnp.zeros((24, H), f32)
    tab = tab.at[:21, :32].set(params["aa_embed"][aa_map])
    tab = tab.at[21:24, 32:].set(params["atom_embed"][atom_map])

    def lin(p):
        return p["w"].astype(f32), p["b"].astype(f32)

    LW = []
    for lay in params["layers"]:
        wm, bm = lin(lay["sake_msg"])
        wa, ba = lin(lay["sake_att"])
        wg, bg = lin(lay["sake_gate"])
        wu, bu = lin(lay["sake_upd"])
        wf1, bf1 = lin(lay["schnet_f1"])
        wf2, bf2 = lin(lay["schnet_f2"])
        wl1, bl1 = lin(lay["schnet_l1"])
        wl2, bl2 = lin(lay["schnet_l2"])
        wl3, bl3 = lin(lay["schnet_l3"])
        wc24 = jnp.zeros((24, H), f32).at[:18].set(wm[128:146])
        wag = jnp.zeros((H, 8), f32).at[:, :4].set(wa).at[:, 4:5].set(wg)
        bag = jnp.zeros((1, 8), f32).at[0, :4].set(ba).at[0, 4].set(bg[0])
        LW.append(dict(
            wa=wm[:64], wb=wm[64:128], wc=wc24, bm=bm[None], wag=wag, bag=bag,
            wuz=wu[:64], wua=wu[64:128], wuv=wu[128:129], bu=bu[None],
            wf1=jnp.zeros((24, H), f32).at[:18].set(wf1), bf1=bf1[None],
            wf2=wf2, bf2=bf2[None],
            wl1=wl1, bl1=bl1[None], wl2=wl2, bl2=bl2[None], wl3=wl3, bl3=bl3[None],
        ))
    we, be = lin(params["embed_out"])
    w1, b1 = lin(params["out"][0])
    w2, b2 = lin(params["out"][1])
    w3, b3 = lin(params["out"][2])
    w3p = jnp.zeros((H, 8), f32).at[:, :1].set(w3)
    b3p = jnp.zeros((1, 8), f32).at[0, 0].set(b3[0])

    PAD = EP - E
    pad_g = (jnp.arange(PAD, dtype=jnp.int32) * 97) % N
    pad_s = N + (jnp.arange(PAD, dtype=jnp.int32) % (N_ACC - N))

    def prep_g(ix):
        return jnp.concatenate([ix.astype(jnp.int32), pad_g]).reshape(EP // 128, 128)

    s_src_g = prep_g(sake_edges[0])
    s_dst_g = prep_g(sake_edges[1])
    c_src_g = prep_g(schnet_edges[0])
    c_dst_g = prep_g(schnet_edges[1])
    s_dst_x = jnp.concatenate([sake_edges[1].astype(jnp.int32), pad_s])
    c_dst_x = jnp.concatenate([schnet_edges[1].astype(jnp.int32), pad_s])

    aa3 = aa_idx.astype(jnp.int32).reshape(GN, 1, BLK_N)
    at3 = atom_idx.astype(jnp.int32).reshape(GN, 1, BLK_N)

    pos16 = jnp.pad(pos, ((0, 0), (0, 13)))
    pn = jnp.concatenate([pos16, -pos16, jnp.zeros((N, 32), f32)], axis=1)

    z, ta, tb = _node0_call(aa3, at3, tab, LW[0]["wa"], LW[0]["wb"], pn)

    for li, W in enumerate(LW):
        zsg = _gathersum_call(ta, tb, s_src_g, s_dst_g, _SMAP_SAKE)
        elgd, bmax = _edgeA_call(zsg, W["wc"], W["bm"], W["wag"], W["bag"])
        p0, p1 = _edgeB_call(elgd, bmax)
        acc0 = jax.ops.segment_sum(p0, s_dst_x, num_segments=N_ACC)
        acc1 = jax.ops.segment_sum(p1, s_dst_x, num_segments=N_ACC)
        zmid, t1 = _nodeC_call(z, acc0, acc1, W["wuz"], W["wua"], W["wuv"],
                               W["bu"], W["wl1"], W["bl1"], pn)
        zc = _gathersum_call(t1, t1, c_src_g, c_dst_g, _SMAP_SCHNET)
        m0, m1 = _edgeS_call(zc, W["wf1"], W["bf1"], W["wf2"], W["bf2"])
        b0 = jax.ops.segment_sum(m0, c_dst_x, num_segments=N_ACC)
        b1_ = jax.ops.segment_sum(m1, c_dst_x, num_segments=N_ACC)
        if li + 1 < len(LW):
            W2 = LW[li + 1]
            z, ta, tb = _nodeE_call(zmid, b0, b1_, W["wl2"], W["bl2"],
                                    W["wl3"], W["bl3"], W2["wa"], W2["wb"], pn)
        else:
            [out8] = _nodeF_call(zmid, b0, b1_, W["wl2"], W["bl2"], W["wl3"],
                                 W["bl3"], we, be[None], w1, b1[None], w2,
                                 b2[None], w3p, b3p)
    return out8[:, :1]
